# baseline (speedup 1.0000x reference)
import functools

import jax
import jax.numpy as jnp
from jax import lax
from jax.experimental import pallas as pl
from jax.experimental.pallas import tpu as pltpu
from jax.experimental.pallas import tpu_sc as plsc

N = 10000
E = 320000
NC = 2
NS = 16
TILES = NC * NS
CH = 125
CPT = 80
PFMAX = 6
NPAD = 10112
RPT = NPAD // NS
EPS = 1e-5

_MESH = plsc.VectorSubcoreMesh(core_axis_name="c", subcore_axis_name="s")
_SC_PARAMS = pltpu.CompilerParams(use_tc_tiling_on_sc=False)


def _make_agg(width, nbuf, lag=2):
    pf = nbuf - lag
    ns = CPT

    @functools.partial(
        pl.kernel,
        out_type=jax.ShapeDtypeStruct((NC, NPAD, width), jnp.float32),
        mesh=_MESH,
        scratch_types=[
            pltpu.VMEM((CPT + pf, CH), jnp.int32),
            pltpu.VMEM((CPT, CH), jnp.int32),
            pltpu.VMEM((nbuf, CH, width), jnp.float32),
            pltpu.VMEM_SHARED((NPAD, width), jnp.float32),
            pltpu.SemaphoreType.DMA,
            pltpu.SemaphoreType.DMA,
        ],
        compiler_params=_SC_PARAMS,
    )
    def agg(table, srcdst, safe, zeros, out, idx_s, idx_d, buf, acc, gsem, ssem):
        c = lax.axis_index("c")
        s = lax.axis_index("s")
        wid = c * NS + s
        r0 = s * RPT
        @pl.when(c == 0)
        def _():
            pltpu.sync_copy(table.at[pl.ds(r0, RPT)], acc.at[pl.ds(r0, RPT)])

        @pl.when(c != 0)
        def _():
            pltpu.sync_copy(zeros, acc.at[pl.ds(r0, RPT)])
        pltpu.sync_copy(srcdst.at[0, wid], idx_s.at[pl.ds(0, CPT)])
        pltpu.sync_copy(safe.at[pl.ds(0, pf)], idx_s.at[pl.ds(CPT, pf)])
        pltpu.sync_copy(srcdst.at[1, wid], idx_d)
        plsc.subcore_barrier()

        def gidx(j):
            return idx_s.at[j]

        def sidx(j):
            return idx_d.at[j]

        for j in range(pf):
            pltpu.async_copy(table.at[gidx(j)], buf.at[j], gsem)
        for j in range(lag):
            pltpu.make_async_copy(table.at[gidx(j)], buf.at[j], gsem).wait()
            pltpu.async_copy(buf.at[j], acc.at[sidx(j)], ssem, add=True)
            pltpu.async_copy(table.at[gidx(j + pf)], buf.at[(j + pf) % nbuf], gsem)

        def slot(j, carry):
            b = jnp.bitwise_and(j, nbuf - 1)
            bml = jnp.bitwise_and(j + pf, nbuf - 1)
            pltpu.make_async_copy(table.at[gidx(j)], buf.at[b], gsem).wait()
            pltpu.async_copy(buf.at[b], acc.at[sidx(j)], ssem, add=True)
            pltpu.make_async_copy(buf.at[bml], acc.at[sidx(j - lag)], ssem).wait()
            pltpu.async_copy(table.at[gidx(j + pf)], buf.at[bml], gsem)
            return carry

        lax.fori_loop(lag, ns, slot, 0)
        for k in range(lag, 0, -1):
            pltpu.make_async_copy(
                buf.at[(ns - k) % nbuf], acc.at[sidx(ns - k)], ssem).wait()
        for k in range(pf):
            pltpu.make_async_copy(
                table.at[gidx(ns + k)], buf.at[(ns + k) % nbuf], gsem
            ).wait()
        plsc.subcore_barrier()
        pltpu.sync_copy(acc.at[pl.ds(r0, RPT)], out.at[c, pl.ds(r0, RPT)])

    return agg


_agg64 = _make_agg(64, 8, lag=3)
_agg16 = _make_agg(16, 8, lag=2)


DW = 8


@functools.partial(
    pl.kernel,
    out_type=jax.ShapeDtypeStruct((NC, NPAD, DW), jnp.float32),
    mesh=_MESH,
    scratch_types=[
        pltpu.VMEM((CPT, CH), jnp.int32),
        pltpu.VMEM((CH, DW), jnp.float32),
        pltpu.VMEM_SHARED((NPAD, DW), jnp.float32),
        pltpu.SemaphoreType.DMA,
    ],
    compiler_params=_SC_PARAMS,
)
def _deg(srcdst, ones, zeros, out, idx_d, obuf, acc, ssem):
    c = lax.axis_index("c")
    s = lax.axis_index("s")
    wid = c * NS + s
    r0 = s * RPT
    pltpu.sync_copy(zeros, acc.at[pl.ds(r0, RPT)])
    pltpu.sync_copy(srcdst.at[1, wid], idx_d)
    pltpu.sync_copy(ones, obuf)
    plsc.subcore_barrier()

    for j in (0, 1):
        pltpu.async_copy(obuf, acc.at[idx_d.at[j]], ssem, add=True)

    def slot(j, carry):
        pltpu.async_copy(obuf, acc.at[idx_d.at[j]], ssem, add=True)
        pltpu.make_async_copy(obuf, acc.at[idx_d.at[j - 2]], ssem).wait()
        return carry

    lax.fori_loop(2, CPT, slot, 0)
    pltpu.make_async_copy(obuf, acc.at[idx_d.at[CPT - 2]], ssem).wait()
    pltpu.make_async_copy(obuf, acc.at[idx_d.at[CPT - 1]], ssem).wait()
    plsc.subcore_barrier()
    pltpu.sync_copy(acc.at[pl.ds(r0, RPT)], out.at[c, pl.ds(r0, RPT)])


def _proj_body(x, w, out):
    out[...] = jnp.dot(x[...], w[...], preferred_element_type=jnp.float32)


_proj = pl.pallas_call(
    _proj_body, out_shape=jax.ShapeDtypeStruct((N, 64), jnp.float32)
)


def _scale1_body(p, h, h_out, dinv_out):
    deg = p[0, :, 0:1] + p[1, :, 0:1] + 1.0
    dinv = lax.rsqrt(deg)
    dinv_out[...] = dinv
    h_out[0:N, :] = h[...] * dinv[0:N]
    h_out[N:NPAD, :] = jnp.zeros((NPAD - N, 64), jnp.float32)


_scale1 = pl.pallas_call(
    _scale1_body,
    out_shape=(
        jax.ShapeDtypeStruct((NPAD, 64), jnp.float32),
        jax.ShapeDtypeStruct((NPAD, 1), jnp.float32),
    ),
)


def _mid_body(p, dinv, b, g, be, w, out):
    t = dinv[0:N] * (p[0, 0:N] + p[1, 0:N]) + b[...]
    mu = jnp.mean(t, axis=0, keepdims=True)
    var = jnp.mean(t * t, axis=0, keepdims=True) - mu * mu
    t = g[...] * (t - mu) * lax.rsqrt(var + EPS) + be[...]
    t = jnp.maximum(t, 0.0)
    h = jnp.dot(t, w[...], preferred_element_type=jnp.float32)
    out[0:N, :] = h * dinv[0:N]
    out[N:NPAD, :] = jnp.zeros((NPAD - N, h.shape[1]), jnp.float32)


def _make_mid(wout):
    return pl.pallas_call(
        _mid_body,
        out_shape=jax.ShapeDtypeStruct((NPAD, wout), jnp.float32),
    )


_mid64 = _make_mid(64)
_mid16 = _make_mid(16)


def _final_body(p, dinv, b, g, be, out):
    t = dinv[0:N] * (p[0, 0:N] + p[1, 0:N])
    t = t[:, 0:2] + b[...]
    mu = jnp.mean(t, axis=0, keepdims=True)
    var = jnp.mean(t * t, axis=0, keepdims=True) - mu * mu
    out[...] = g[...] * (t - mu) * lax.rsqrt(var + EPS) + be[...]


_final = pl.pallas_call(
    _final_body,
    out_shape=jax.ShapeDtypeStruct((N, 2), jnp.float32),
)


def kernel(x, edge_index, W1, b1, g1, be1, W2, b2, g2, be2, W3, b3, g3, be3):
    srcdst = edge_index.astype(jnp.int32).reshape(2, TILES, CPT, CH)
    safe = (N + jnp.arange(PFMAX * CH, dtype=jnp.int32)
            % (NPAD - N)).reshape(PFMAX, CH)

    ones = jnp.ones((CH, DW), jnp.float32)
    zd = jnp.zeros((RPT, DW), jnp.float32)
    z16 = jnp.zeros((RPT, 16), jnp.float32)
    z64 = jnp.zeros((RPT, 64), jnp.float32)

    pdeg = _deg(srcdst, ones, zd)
    h1 = _proj(x, W1)
    h1p, dinv = _scale1(pdeg, h1)

    p1 = _agg64(h1p, srcdst, safe, z64)
    h2p = _mid64(p1, dinv,
                 b1.reshape(1, -1), g1.reshape(1, -1), be1.reshape(1, -1), W2)

    p2 = _agg64(h2p, srcdst, safe, z64)
    W3p = jnp.pad(W3, ((0, 0), (0, 16 - W3.shape[1])))
    h3p = _mid16(p2, dinv,
                 b2.reshape(1, -1), g2.reshape(1, -1), be2.reshape(1, -1), W3p)

    p3 = _agg16(h3p, srcdst, safe, z16)
    out = _final(p3, dinv,
                 b3.reshape(1, -1), g3.reshape(1, -1), be3.reshape(1, -1))
    return out


# speedup vs baseline: 1.0092x; 1.0092x over previous
import functools

import jax
import jax.numpy as jnp
from jax import lax
from jax.experimental import pallas as pl
from jax.experimental.pallas import tpu as pltpu
from jax.experimental.pallas import tpu_sc as plsc

N = 10000
E = 320000
NC = 2
NS = 16
TILES = NC * NS
CH = 125
CPT = 80
PFMAX = 6
NPAD = 10112
RPT = NPAD // NS
EPS = 1e-5

_MESH = plsc.VectorSubcoreMesh(core_axis_name="c", subcore_axis_name="s")
_SC_PARAMS = pltpu.CompilerParams(use_tc_tiling_on_sc=False)


def _make_agg(width, nbuf, lag=2):
    pf = nbuf - lag
    ns = CPT

    @functools.partial(
        pl.kernel,
        out_type=jax.ShapeDtypeStruct((NC, NPAD, width), jnp.float32),
        mesh=_MESH,
        scratch_types=[
            pltpu.VMEM((CPT + pf, CH), jnp.int32),
            pltpu.VMEM((CPT, CH), jnp.int32),
            pltpu.VMEM((nbuf, CH, width), jnp.float32),
            pltpu.VMEM_SHARED((NPAD, width), jnp.float32),
            pltpu.SemaphoreType.DMA,
            pltpu.SemaphoreType.DMA,
        ],
        compiler_params=_SC_PARAMS,
    )
    def agg(table, srcdst, safe, zeros, out, idx_s, idx_d, buf, acc, gsem, ssem):
        c = lax.axis_index("c")
        s = lax.axis_index("s")
        wid = c * NS + s
        r0 = s * RPT
        @pl.when(c == 0)
        def _():
            pltpu.sync_copy(table.at[pl.ds(r0, RPT)], acc.at[pl.ds(r0, RPT)])

        @pl.when(c != 0)
        def _():
            pltpu.sync_copy(zeros, acc.at[pl.ds(r0, RPT)])
        pltpu.sync_copy(srcdst.at[0, wid], idx_s.at[pl.ds(0, CPT)])
        pltpu.sync_copy(safe.at[pl.ds(0, pf)], idx_s.at[pl.ds(CPT, pf)])
        pltpu.sync_copy(srcdst.at[1, wid], idx_d)
        plsc.subcore_barrier()

        def gidx(j):
            return idx_s.at[j]

        def sidx(j):
            return idx_d.at[j]

        for j in range(pf):
            pltpu.async_copy(table.at[gidx(j)], buf.at[j], gsem)
        for j in range(lag):
            pltpu.make_async_copy(table.at[gidx(j)], buf.at[j], gsem).wait()
            pltpu.async_copy(buf.at[j], acc.at[sidx(j)], ssem, add=True)
            pltpu.async_copy(table.at[gidx(j + pf)], buf.at[(j + pf) % nbuf], gsem)

        def slot(j, carry):
            b = jnp.bitwise_and(j, nbuf - 1)
            bml = jnp.bitwise_and(j + pf, nbuf - 1)
            pltpu.make_async_copy(table.at[gidx(j)], buf.at[b], gsem).wait()
            pltpu.async_copy(buf.at[b], acc.at[sidx(j)], ssem, add=True)
            pltpu.make_async_copy(buf.at[bml], acc.at[sidx(j - lag)], ssem).wait()
            pltpu.async_copy(table.at[gidx(j + pf)], buf.at[bml], gsem)
            return carry

        lax.fori_loop(lag, ns, slot, 0)
        for k in range(lag, 0, -1):
            pltpu.make_async_copy(
                buf.at[(ns - k) % nbuf], acc.at[sidx(ns - k)], ssem).wait()
        for k in range(pf):
            pltpu.make_async_copy(
                table.at[gidx(ns + k)], buf.at[(ns + k) % nbuf], gsem
            ).wait()
        plsc.subcore_barrier()
        pltpu.sync_copy(acc.at[pl.ds(r0, RPT)], out.at[c, pl.ds(r0, RPT)])

    return agg


_agg64 = _make_agg(64, 4, lag=2)
_agg16 = _make_agg(16, 8, lag=2)


DW = 8


@functools.partial(
    pl.kernel,
    out_type=jax.ShapeDtypeStruct((NC, NPAD, DW), jnp.float32),
    mesh=_MESH,
    scratch_types=[
        pltpu.VMEM((CPT, CH), jnp.int32),
        pltpu.VMEM((CH, DW), jnp.float32),
        pltpu.VMEM_SHARED((NPAD, DW), jnp.float32),
        pltpu.SemaphoreType.DMA,
    ],
    compiler_params=_SC_PARAMS,
)
def _deg(srcdst, ones, zeros, out, idx_d, obuf, acc, ssem):
    c = lax.axis_index("c")
    s = lax.axis_index("s")
    wid = c * NS + s
    r0 = s * RPT
    pltpu.sync_copy(zeros, acc.at[pl.ds(r0, RPT)])
    pltpu.sync_copy(srcdst.at[1, wid], idx_d)
    pltpu.sync_copy(ones, obuf)
    plsc.subcore_barrier()

    for j in (0, 1):
        pltpu.async_copy(obuf, acc.at[idx_d.at[j]], ssem, add=True)

    def slot(j, carry):
        pltpu.async_copy(obuf, acc.at[idx_d.at[j]], ssem, add=True)
        pltpu.make_async_copy(obuf, acc.at[idx_d.at[j - 2]], ssem).wait()
        return carry

    lax.fori_loop(2, CPT, slot, 0)
    pltpu.make_async_copy(obuf, acc.at[idx_d.at[CPT - 2]], ssem).wait()
    pltpu.make_async_copy(obuf, acc.at[idx_d.at[CPT - 1]], ssem).wait()
    plsc.subcore_barrier()
    pltpu.sync_copy(acc.at[pl.ds(r0, RPT)], out.at[c, pl.ds(r0, RPT)])


def _proj_body(x, w, out):
    out[...] = jnp.dot(x[...], w[...], preferred_element_type=jnp.float32)


_proj = pl.pallas_call(
    _proj_body, out_shape=jax.ShapeDtypeStruct((N, 64), jnp.float32)
)


def _scale1_body(p, h, h_out, dinv_out):
    deg = p[0, :, 0:1] + p[1, :, 0:1] + 1.0
    dinv = lax.rsqrt(deg)
    dinv_out[...] = dinv
    h_out[0:N, :] = h[...] * dinv[0:N]
    h_out[N:NPAD, :] = jnp.zeros((NPAD - N, 64), jnp.float32)


_scale1 = pl.pallas_call(
    _scale1_body,
    out_shape=(
        jax.ShapeDtypeStruct((NPAD, 64), jnp.float32),
        jax.ShapeDtypeStruct((NPAD, 1), jnp.float32),
    ),
)


def _mid_body(p, dinv, b, g, be, w, out):
    t = dinv[0:N] * (p[0, 0:N] + p[1, 0:N]) + b[...]
    mu = jnp.mean(t, axis=0, keepdims=True)
    var = jnp.mean(t * t, axis=0, keepdims=True) - mu * mu
    t = g[...] * (t - mu) * lax.rsqrt(var + EPS) + be[...]
    t = jnp.maximum(t, 0.0)
    h = jnp.dot(t, w[...], preferred_element_type=jnp.float32)
    out[0:N, :] = h * dinv[0:N]
    out[N:NPAD, :] = jnp.zeros((NPAD - N, h.shape[1]), jnp.float32)


def _make_mid(wout):
    return pl.pallas_call(
        _mid_body,
        out_shape=jax.ShapeDtypeStruct((NPAD, wout), jnp.float32),
    )


_mid64 = _make_mid(64)
_mid16 = _make_mid(16)


def _final_body(p, dinv, b, g, be, out):
    t = dinv[0:N] * (p[0, 0:N] + p[1, 0:N])
    t = t[:, 0:2] + b[...]
    mu = jnp.mean(t, axis=0, keepdims=True)
    var = jnp.mean(t * t, axis=0, keepdims=True) - mu * mu
    out[...] = g[...] * (t - mu) * lax.rsqrt(var + EPS) + be[...]


_final = pl.pallas_call(
    _final_body,
    out_shape=jax.ShapeDtypeStruct((N, 2), jnp.float32),
)


def kernel(x, edge_index, W1, b1, g1, be1, W2, b2, g2, be2, W3, b3, g3, be3):
    srcdst = edge_index.astype(jnp.int32).reshape(2, TILES, CPT, CH)
    safe = (N + jnp.arange(PFMAX * CH, dtype=jnp.int32)
            % (NPAD - N)).reshape(PFMAX, CH)

    ones = jnp.ones((CH, DW), jnp.float32)
    zd = jnp.zeros((RPT, DW), jnp.float32)
    z16 = jnp.zeros((RPT, 16), jnp.float32)
    z64 = jnp.zeros((RPT, 64), jnp.float32)

    pdeg = _deg(srcdst, ones, zd)
    h1 = _proj(x, W1)
    h1p, dinv = _scale1(pdeg, h1)

    p1 = _agg64(h1p, srcdst, safe, z64)
    h2p = _mid64(p1, dinv,
                 b1.reshape(1, -1), g1.reshape(1, -1), be1.reshape(1, -1), W2)

    p2 = _agg64(h2p, srcdst, safe, z64)
    W3p = jnp.pad(W3, ((0, 0), (0, 16 - W3.shape[1])))
    h3p = _mid16(p2, dinv,
                 b2.reshape(1, -1), g2.reshape(1, -1), be2.reshape(1, -1), W3p)

    p3 = _agg16(h3p, srcdst, safe, z16)
    out = _final(p3, dinv,
                 b3.reshape(1, -1), g3.reshape(1, -1), be3.reshape(1, -1))
    return out
